# baseline (speedup 1.0000x reference)
import functools

import jax
import jax.numpy as jnp
import numpy as np
from jax import lax
from jax.experimental import pallas as pl
from jax.experimental.pallas import tpu as pltpu
from jax.experimental.pallas import tpu_sc as plsc

_NUM_FEATURES = 26
_ROWS_PER_FEATURE = 100000
_CHUNK = 128


@functools.cache
def _build(batch, feat, dim, nw):
    rows_per_w = batch // nw
    n_chunks = rows_per_w // _CHUNK
    chf = _CHUNK * feat
    mesh = plsc.VectorSubcoreMesh(core_axis_name="c", subcore_axis_name="s")

    @functools.partial(
        pl.kernel,
        out_type=jax.ShapeDtypeStruct((batch, dim), jnp.float32),
        mesh=mesh,
        compiler_params=pltpu.CompilerParams(use_tc_tiling_on_sc=False),
        scratch_types=[
            pltpu.VMEM((chf,), jnp.int32),
            pltpu.VMEM((chf,), jnp.int32),
            pltpu.VMEM((chf, dim), jnp.float32),
            pltpu.VMEM((_CHUNK, dim), jnp.float32),
            pltpu.SemaphoreType.DMA,
        ],
    )
    def k(x_hbm, offs_hbm, table_hbm, out_hbm, idx_v, offs_v, rows_v, acc_v, sem):
        wid = lax.axis_index("s") * 2 + lax.axis_index("c")
        base = wid * rows_per_w
        pltpu.sync_copy(offs_hbm, offs_v)

        def chunk_body(c, carry):
            cb = base + c * _CHUNK
            pltpu.sync_copy(x_hbm.at[pl.ds(cb * feat, chf)], idx_v)

            def add_body(i, carry2):
                s = i * 16
                idx_v[pl.ds(s, 16)] = idx_v[pl.ds(s, 16)] + offs_v[pl.ds(s, 16)]
                return carry2

            lax.fori_loop(0, chf // 16, add_body, 0, unroll=8)

            descs = []
            for j in range(feat):
                descs.append(
                    pltpu.async_copy(
                        table_hbm.at[idx_v.at[pl.ds(j * _CHUNK, _CHUNK)]],
                        rows_v.at[pl.ds(j * _CHUNK, _CHUNK)],
                        sem,
                    )
                )
            for d in descs:
                d.wait()

            def sum_rows(b, carry3):
                a = rows_v.at[b * feat][...]
                for j in range(1, feat):
                    a = a + rows_v.at[b * feat + j][...]
                acc_v.at[b][...] = a
                return carry3

            lax.fori_loop(0, _CHUNK, sum_rows, 0)
            pltpu.sync_copy(acc_v, out_hbm.at[pl.ds(cb, _CHUNK)])
            return carry

        lax.fori_loop(0, n_chunks, chunk_body, 0)

    return k


def kernel(x, table):
    batch, feat = x.shape
    dim = table.shape[1]
    info = plsc.get_sparse_core_info()
    nw = info.num_cores * info.num_subcores
    offsets = np.arange(feat, dtype=np.int32) * _ROWS_PER_FEATURE
    offs_rep = jnp.asarray(np.tile(offsets, _CHUNK))
    x_flat = x.reshape(-1).astype(jnp.int32)
    return _build(batch, feat, dim, nw)(x_flat, offs_rep, table)


# speedup vs baseline: 1.0016x; 1.0016x over previous
import functools

import jax
import jax.numpy as jnp
import numpy as np
from jax import lax
from jax.experimental import pallas as pl
from jax.experimental.pallas import tpu as pltpu
from jax.experimental.pallas import tpu_sc as plsc

_NUM_FEATURES = 26
_ROWS_PER_FEATURE = 100000
_CHUNK = 128


@functools.cache
def _build(batch, feat, dim, nw):
    rows_per_w = batch // nw
    n_chunks = rows_per_w // _CHUNK
    chf = _CHUNK * feat
    mesh = plsc.VectorSubcoreMesh(core_axis_name="c", subcore_axis_name="s")

    @functools.partial(
        pl.kernel,
        out_type=jax.ShapeDtypeStruct((batch, dim), jnp.float32),
        mesh=mesh,
        compiler_params=pltpu.CompilerParams(use_tc_tiling_on_sc=False),
        scratch_types=[
            pltpu.VMEM((chf,), jnp.int32),
            pltpu.VMEM((chf,), jnp.int32),
            pltpu.VMEM((chf, dim), jnp.float32),
            pltpu.VMEM((_CHUNK, dim), jnp.float32),
            pltpu.SemaphoreType.DMA,
        ],
    )
    def k(x_hbm, offs_hbm, table_hbm, out_hbm, idx_v, offs_v, rows_v, acc_v, sem):
        wid = lax.axis_index("s") * 2 + lax.axis_index("c")
        base = wid * rows_per_w
        pltpu.sync_copy(offs_hbm, offs_v)

        def chunk_body(c, carry):
            cb = base + c * _CHUNK
            pltpu.sync_copy(x_hbm.at[pl.ds(cb * feat, chf)], idx_v)

            def add_body(i, carry2):
                s = i * 16
                idx_v[pl.ds(s, 16)] = idx_v[pl.ds(s, 16)] + offs_v[pl.ds(s, 16)]
                return carry2

            lax.fori_loop(0, chf // 16, add_body, 0, unroll=8)

            pltpu.async_copy(table_hbm.at[idx_v], rows_v, sem).wait()

            def sum_rows(b, carry3):
                a = rows_v.at[b * feat][...]
                for j in range(1, feat):
                    a = a + rows_v.at[b * feat + j][...]
                acc_v.at[b][...] = a
                return carry3

            lax.fori_loop(0, _CHUNK, sum_rows, 0)
            pltpu.sync_copy(acc_v, out_hbm.at[pl.ds(cb, _CHUNK)])
            return carry

        lax.fori_loop(0, n_chunks, chunk_body, 0)

    return k


def kernel(x, table):
    batch, feat = x.shape
    dim = table.shape[1]
    info = plsc.get_sparse_core_info()
    nw = info.num_cores * info.num_subcores
    offsets = np.arange(feat, dtype=np.int32) * _ROWS_PER_FEATURE
    offs_rep = jnp.asarray(np.tile(offsets, _CHUNK))
    x_flat = x.reshape(-1).astype(jnp.int32)
    return _build(batch, feat, dim, nw)(x_flat, offs_rep, table)
